# baseline (speedup 1.0000x reference)
import jax
import jax.numpy as jnp
from jax.experimental import pallas as pl

N_VEH = 6
N_PED = 2
LOGIT_DIM = 6

_B, _N, _D = 32, 256, 1024
_H = _D // 2
_T = _B * _N
_BLK = 256

_SQRT_HALF = 0.7071067811865476


def _body(x_ref, t_ref, w1v_ref, b1v_ref, w2v_ref, b2v_ref,
          w1p_ref, b1p_ref, w2p_ref, b2p_ref,
          out_ref, mv_ref, mp_ref):
    x = x_ref[...]
    t = t_ref[...]
    mv = t == 0
    mp = t == 1

    gv = jnp.dot(x, w1v_ref[...], preferred_element_type=jnp.float32) + b1v_ref[...]
    hv = 0.5 * gv * (1.0 + jax.lax.erf(gv * _SQRT_HALF))
    gp = jnp.dot(x, w1p_ref[...], preferred_element_type=jnp.float32) + b1p_ref[...]
    hp = 0.5 * gp * (1.0 + jax.lax.erf(gp * _SQRT_HALF))

    hv = hv * mv.astype(jnp.float32)
    hp = hp * mp.astype(jnp.float32)
    out = (jnp.dot(hv, w2v_ref[...], preferred_element_type=jnp.float32)
           + jnp.dot(hp, w2p_ref[...], preferred_element_type=jnp.float32))
    out = out + jnp.where(mv, b2v_ref[...], 0.0) + jnp.where(mp, b2p_ref[...], 0.0)
    out_ref[...] = out
    mv_ref[...] = mv
    mp_ref[...] = mp


def kernel(repr3, agent_type_ids, W1v, b1v, W2v, b2v, W1p, b1p, W2p, b2p):
    x = repr3.reshape(_T, _D)
    t = agent_type_ids.reshape(_T, 1)

    w2p6 = jnp.pad(W2p, ((0, 0), (0, LOGIT_DIM - N_PED)))
    b1v_r = b1v.reshape(1, _H)
    b1p_r = b1p.reshape(1, _H)
    b2v_r = b2v.reshape(1, LOGIT_DIM)
    b2p_r = jnp.pad(b2p, (0, LOGIT_DIM - N_PED)).reshape(1, LOGIT_DIM)

    nblk = _T // _BLK
    full = lambda i: (0, 0)
    out, mv, mp = pl.pallas_call(
        _body,
        grid=(nblk,),
        in_specs=[
            pl.BlockSpec((_BLK, _D), lambda i: (i, 0)),
            pl.BlockSpec((_BLK, 1), lambda i: (i, 0)),
            pl.BlockSpec((_D, _H), full),
            pl.BlockSpec((1, _H), full),
            pl.BlockSpec((_H, LOGIT_DIM), full),
            pl.BlockSpec((1, LOGIT_DIM), full),
            pl.BlockSpec((_D, _H), full),
            pl.BlockSpec((1, _H), full),
            pl.BlockSpec((_H, LOGIT_DIM), full),
            pl.BlockSpec((1, LOGIT_DIM), full),
        ],
        out_specs=[
            pl.BlockSpec((_BLK, LOGIT_DIM), lambda i: (i, 0)),
            pl.BlockSpec((_BLK, 1), lambda i: (i, 0)),
            pl.BlockSpec((_BLK, 1), lambda i: (i, 0)),
        ],
        out_shape=[
            jax.ShapeDtypeStruct((_T, LOGIT_DIM), jnp.float32),
            jax.ShapeDtypeStruct((_T, 1), jnp.bool_),
            jax.ShapeDtypeStruct((_T, 1), jnp.bool_),
        ],
    )(x, t, W1v, b1v_r, W2v, b2v_r, W1p, b1p_r, w2p6, b2p_r)

    return (out.reshape(_B, _N, LOGIT_DIM),
            mv.reshape(_B, _N),
            mp.reshape(_B, _N))


# speedup vs baseline: 1.2566x; 1.2566x over previous
import jax
import jax.numpy as jnp
from jax.experimental import pallas as pl

N_VEH = 6
N_PED = 2
LOGIT_DIM = 6

_B, _N, _D = 32, 256, 1024
_H = _D // 2
_T = _B * _N
_BLK = 1024

_SQRT_HALF = 0.7071067811865476


def _body(x_ref, t_ref, w1v_ref, b1v_ref, w2v_ref, b2v_ref,
          w1p_ref, b1p_ref, w2p_ref, b2p_ref,
          out_ref, mv_ref, mp_ref):
    x = x_ref[...]
    t = t_ref[...]
    mv = t == 0
    mp = t == 1

    gv = jnp.dot(x, w1v_ref[...], preferred_element_type=jnp.float32) + b1v_ref[...]
    hv = 0.5 * gv * (1.0 + jax.lax.erf(gv * _SQRT_HALF))
    gp = jnp.dot(x, w1p_ref[...], preferred_element_type=jnp.float32) + b1p_ref[...]
    hp = 0.5 * gp * (1.0 + jax.lax.erf(gp * _SQRT_HALF))

    hv = hv * mv.astype(jnp.float32)
    hp = hp * mp.astype(jnp.float32)
    out = (jnp.dot(hv, w2v_ref[...], preferred_element_type=jnp.float32)
           + jnp.dot(hp, w2p_ref[...], preferred_element_type=jnp.float32))
    out = out + jnp.where(mv, b2v_ref[...], 0.0) + jnp.where(mp, b2p_ref[...], 0.0)
    out_ref[...] = out
    mv_ref[...] = mv
    mp_ref[...] = mp


def kernel(repr3, agent_type_ids, W1v, b1v, W2v, b2v, W1p, b1p, W2p, b2p):
    x = repr3.reshape(_T, _D)
    t = agent_type_ids.reshape(_T, 1)

    w2p6 = jnp.pad(W2p, ((0, 0), (0, LOGIT_DIM - N_PED)))
    b1v_r = b1v.reshape(1, _H)
    b1p_r = b1p.reshape(1, _H)
    b2v_r = b2v.reshape(1, LOGIT_DIM)
    b2p_r = jnp.pad(b2p, (0, LOGIT_DIM - N_PED)).reshape(1, LOGIT_DIM)

    nblk = _T // _BLK
    full = lambda i: (0, 0)
    out, mv, mp = pl.pallas_call(
        _body,
        grid=(nblk,),
        in_specs=[
            pl.BlockSpec((_BLK, _D), lambda i: (i, 0)),
            pl.BlockSpec((_BLK, 1), lambda i: (i, 0)),
            pl.BlockSpec((_D, _H), full),
            pl.BlockSpec((1, _H), full),
            pl.BlockSpec((_H, LOGIT_DIM), full),
            pl.BlockSpec((1, LOGIT_DIM), full),
            pl.BlockSpec((_D, _H), full),
            pl.BlockSpec((1, _H), full),
            pl.BlockSpec((_H, LOGIT_DIM), full),
            pl.BlockSpec((1, LOGIT_DIM), full),
        ],
        out_specs=[
            pl.BlockSpec((_BLK, LOGIT_DIM), lambda i: (i, 0)),
            pl.BlockSpec((_BLK, 1), lambda i: (i, 0)),
            pl.BlockSpec((_BLK, 1), lambda i: (i, 0)),
        ],
        out_shape=[
            jax.ShapeDtypeStruct((_T, LOGIT_DIM), jnp.float32),
            jax.ShapeDtypeStruct((_T, 1), jnp.bool_),
            jax.ShapeDtypeStruct((_T, 1), jnp.bool_),
        ],
    )(x, t, W1v, b1v_r, W2v, b2v_r, W1p, b1p_r, w2p6, b2p_r)

    return (out.reshape(_B, _N, LOGIT_DIM),
            mv.reshape(_B, _N),
            mp.reshape(_B, _N))
